# baseline (speedup 1.0000x reference)
import jax
import jax.numpy as jnp
from jax import lax
from jax.experimental import pallas as pl
from jax.experimental.pallas import tpu as pltpu
from jax.experimental.pallas import tpu_sc as plsc

VOCAB = 1000000
D = 64
NI, NJ = 4096, 200
B = NI * NJ
NC, NS = 2, 16
NW = NC * NS

W1 = 1024
G1 = (VOCAB + 2 * W1 - 1) // (2 * W1)
TPACK_ROWS = G1 * W1


def _repack_body(a_ref, b_ref, o_ref):
    ta = jnp.transpose(a_ref[...])
    tb = jnp.transpose(b_ref[...])
    o_ref[...] = jnp.concatenate([ta, tb], axis=1)


_repack = pl.pallas_call(
    _repack_body,
    grid=(G1,),
    in_specs=[
        pl.BlockSpec((D, W1), lambda i: (0, 2 * i)),
        pl.BlockSpec((D, W1), lambda i: (0, jnp.minimum(2 * i + 1, 975))),
    ],
    out_specs=pl.BlockSpec((W1, 128), lambda i: (i, 0)),
    out_shape=jax.ShapeDtypeStruct((TPACK_ROWS, 128), jnp.float32),
)

CHUNK = 1024


def _gather_body(idx_hbm, tview_hbm, out_hbm, idx_v, rows_v, sem):
    nrows = idx_hbm.shape[0]
    per_w = nrows // NW
    n_chunks = per_w // CHUNK
    wid = lax.axis_index("s") * NC + lax.axis_index("c")
    base = wid * per_w

    def step(i, carry):
        off = base + i * CHUNK
        pltpu.sync_copy(idx_hbm.at[pl.ds(off, CHUNK)], idx_v)
        pltpu.async_copy(tview_hbm.at[idx_v], rows_v, sem).wait()
        pltpu.sync_copy(rows_v, out_hbm.at[pl.ds(off, CHUNK)])
        return carry

    lax.fori_loop(0, n_chunks, step, 0)


def _make_gather(nrows):
    return pl.kernel(
        _gather_body,
        out_type=jax.ShapeDtypeStruct((nrows, D), jnp.float32),
        mesh=plsc.VectorSubcoreMesh(core_axis_name="c", subcore_axis_name="s"),
        compiler_params=pltpu.CompilerParams(use_tc_tiling_on_sc=False),
        scratch_types=[
            pltpu.VMEM((CHUNK,), jnp.int32),
            pltpu.VMEM((CHUNK, D), jnp.float32),
            pltpu.SemaphoreType.DMA,
        ],
    )


NQ = NI // 2
NJ_A, NJ_B = 96, 104
ROWS_A, ROWS_B = NJ_A * NI, NJ_B * NI


def _select_t_body_a(r_ref, o_ref):
    blk = r_ref[...]
    tblk = jnp.transpose(blk, (0, 2, 1))
    o_ref[...] = jnp.concatenate([tblk[:, :D, :], tblk[:, D:, :]], axis=2)


def _select_t_body_b(r_ref, _prev_ref, o_ref):
    _select_t_body_a(r_ref, o_ref)


_OUT3_TYPE = jax.ShapeDtypeStruct((NJ, D, NI), jnp.float32)

_select_a = pl.pallas_call(
    _select_t_body_a,
    grid=(NJ_A,),
    in_specs=[pl.BlockSpec((1, NQ, 128), lambda a: (a, 0, 0))],
    out_specs=pl.BlockSpec((1, D, NI), lambda a: (a, 0, 0)),
    out_shape=_OUT3_TYPE,
)

_select_b = pl.pallas_call(
    _select_t_body_b,
    grid=(NJ_B,),
    in_specs=[
        pl.BlockSpec((1, NQ, 128), lambda a: (a, 0, 0)),
        pl.BlockSpec(memory_space=pl.ANY),
    ],
    out_specs=pl.BlockSpec((1, D, NI), lambda a: (a + NJ_A, 0, 0)),
    out_shape=_OUT3_TYPE,
    input_output_aliases={1: 0},
)


def kernel(x, table):
    tT = jnp.transpose(table)
    xt = jnp.transpose(x).astype(jnp.int32)
    tpack = _repack(tT, tT)
    tview = jnp.reshape(tpack, (2 * TPACK_ROWS, D))

    g = ((xt >> 11) << 11) + ((xt & 1023) << 1) + ((xt >> 10) & 1)
    idx3 = jnp.transpose(g.reshape(NJ, 2, NQ), (0, 2, 1)).reshape(-1)

    idx_a = lax.slice(idx3, (0,), (ROWS_A,))
    out2_a = _make_gather(ROWS_A)(idx_a, tview)
    r3_a = jnp.reshape(out2_a, (NJ_A, NQ, 128))

    idx_b = lax.slice(idx3, (ROWS_A,), (B,))
    out2_b = _make_gather(ROWS_B)(idx_b, tview)
    r3_b = jnp.reshape(out2_b, (NJ_B, NQ, 128))

    buf_a = _select_a(r3_a)
    out3 = _select_b(r3_b, buf_a)
    return jnp.transpose(out3, (2, 0, 1))
